# baseline (speedup 1.0000x reference)
import numpy as np
import jax
import jax.numpy as jnp
from jax import lax
from jax.experimental import pallas as pl

_N = 512

_rng = np.random.default_rng(0)
_off = _rng.integers(-2, 0, size=2)
_CY = _N // 2 + int(_off[0])
_CX = _N // 2 + int(_off[1])
_MAXR = int((_N // 2 - 1) * 0.6)
_MINR = int((_N // 2 - 1) * 0.1)
_L = int(_rng.integers(_MINR, _MAXR))
_S = int(_rng.integers(0, _L))
_L2 = _L * _L
_S2 = _S * _S

_ROW_LO, _ROW_HI = _CY - _L + 1, _CY + _L - 1
_COL_LO, _COL_HI = _CX - _L + 1, _CX + _L - 1

_BR = 32
_BC = 128
_NJ = _N // _BR
_JLO, _JHI = _ROW_LO // _BR, _ROW_HI // _BR
_KLO, _KHI = _COL_LO // _BC, _COL_HI // _BC
_NK = _KHI - _KLO + 1


def _body(*refs):
    in_refs, out_ref = refs[:-1], refs[-1]
    j = pl.program_id(1)
    r = j * _BR + lax.broadcasted_iota(jnp.int32, (_BR, _BC), 0)
    dr2 = (r - _CY) ** 2
    if _KLO > 0:
        out_ref[0, :, 0:_KLO * _BC] = jnp.zeros((_BR, _KLO * _BC), jnp.float32)
    for t in range(_NK):
        k = _KLO + t
        c = k * _BC + lax.broadcasted_iota(jnp.int32, (_BR, _BC), 1)
        d2 = dr2 + (c - _CX) ** 2
        m = (d2 < _L2) & (d2 >= _S2)
        out_ref[0, :, k * _BC:(k + 1) * _BC] = jnp.where(m, in_refs[t][0], 0.0)
    if _KHI + 1 < _N // _BC:
        out_ref[0, :, (_KHI + 1) * _BC:] = jnp.zeros(
            (_BR, _N - (_KHI + 1) * _BC), jnp.float32)


def _make_call(nimg, interpret=False):
    def _in_spec(k):
        return pl.BlockSpec(
            (1, _BR, _BC),
            lambda i, j, _k=k: (i, jnp.clip(j, _JLO, _JHI), _k),
        )

    return pl.pallas_call(
        _body,
        grid=(nimg, _NJ),
        in_specs=[_in_spec(_KLO + t) for t in range(_NK)],
        out_specs=pl.BlockSpec((1, _BR, _N), lambda i, j: (i, j, 0)),
        out_shape=jax.ShapeDtypeStruct((nimg, _N, _N), jnp.float32),
        interpret=interpret,
    )


def kernel(x):
    nimg = x.shape[0] * x.shape[1]
    xr = x.reshape(nimg, _N, _N)
    y = _make_call(nimg)(*([xr] * _NK))
    return y.reshape(x.shape)


# speedup vs baseline: 16.1959x; 16.1959x over previous
import numpy as np
import jax
import jax.numpy as jnp
from jax import lax
from jax.experimental import pallas as pl

_N = 512

_rng = np.random.default_rng(0)
_off = _rng.integers(-2, 0, size=2)
_CY = _N // 2 + int(_off[0])
_CX = _N // 2 + int(_off[1])
_MAXR = int((_N // 2 - 1) * 0.6)
_MINR = int((_N // 2 - 1) * 0.1)
_L = int(_rng.integers(_MINR, _MAXR))
_S = int(_rng.integers(0, _L))
_L2 = _L * _L
_S2 = _S * _S

_R0 = ((_CY - _L + 1) // 8) * 8
_R1 = -((-(_CY + _L)) // 8) * 8
_C0 = ((_CX - _L + 1) // 128) * 128
_C1 = -((-(_CX + _L)) // 128) * 128
_BBR = _R1 - _R0
_BBC = _C1 - _C0

_IPB = 8
_NIMG = 96


def _body(in_ref, out_ref):
    r = _R0 + lax.broadcasted_iota(jnp.int32, (_BBR, _BBC), 0)
    c = _C0 + lax.broadcasted_iota(jnp.int32, (_BBR, _BBC), 1)
    d2 = (r - _CY) ** 2 + (c - _CX) ** 2
    m = (d2 < _L2) & (d2 >= _S2)
    out_ref[:, 0:_R0, :] = jnp.zeros((_IPB, _R0, _N), jnp.float32)
    out_ref[:, _R1:_N, :] = jnp.zeros((_IPB, _N - _R1, _N), jnp.float32)
    out_ref[:, _R0:_R1, 0:_C0] = jnp.zeros((_IPB, _BBR, _C0), jnp.float32)
    out_ref[:, _R0:_R1, _C1:_N] = jnp.zeros((_IPB, _BBR, _N - _C1), jnp.float32)
    out_ref[:, _R0:_R1, _C0:_C1] = jnp.where(m[None], in_ref[...], 0.0)


def _make_call(interpret=False):
    return pl.pallas_call(
        _body,
        grid=(_NIMG // _IPB,),
        in_specs=[pl.BlockSpec(
            (pl.Element(_IPB), pl.Element(_BBR), pl.Element(_BBC)),
            lambda i: (i * _IPB, _R0, _C0),
        )],
        out_specs=pl.BlockSpec((_IPB, _N, _N), lambda i: (i, 0, 0)),
        out_shape=jax.ShapeDtypeStruct((_NIMG, _N, _N), jnp.float32),
        interpret=interpret,
    )


def kernel(x):
    xr = x.reshape(_NIMG, _N, _N)
    y = _make_call()(xr)
    return y.reshape(x.shape)
